# baseline (speedup 1.0000x reference)
import jax
import jax.numpy as jnp
import numpy as np
from jax.experimental import pallas as pl

D_IN = 1024
D_SAE = 4096
T = 8
K = 32
RB = 256
MAX_ITERS = 64


def _count_ge(pre, c):
    return jnp.sum((pre >= c).astype(jnp.int32), axis=1, keepdims=True)


def _encode_kernel(x_ref, We_ref, be_ref, bd_ref, z_ref):
    xc = x_ref[...] - bd_ref[0]
    pre = jax.lax.dot_general(
        xc, We_ref[0],
        (((1,), (0,)), ((), ())),
        preferred_element_type=jnp.float32,
    ) + be_ref[0]

    maxv = jnp.max(pre, axis=1, keepdims=True)
    c_hi = _count_ge(pre, maxv)
    c_lo = _count_ge(pre, 0.0)

    done = ((c_lo <= K) | (c_hi >= K) | (maxv <= 0.0)).astype(jnp.int32)
    thr = jnp.where(c_hi >= K, maxv,
                    jnp.where(maxv <= 0.0, jnp.float32(1.0),
                              jnp.float32(0.0)))
    v_lo = jnp.zeros_like(maxv)
    v_hi = maxv
    it0 = jnp.int32(0)

    def cond(state):
        it, _, _, _, _, _, done_i = state
        return jnp.logical_and(it < MAX_ITERS, jnp.min(done_i) < 1)

    def body(state):
        it, v_lo, v_hi, c_lo, c_hi, thr, done_i = state
        done = done_i > 0
        s_lo = jax.lax.bitcast_convert_type(v_lo, jnp.int32)
        s_hi = jax.lax.bitcast_convert_type(v_hi, jnp.int32)
        v_bis = jax.lax.bitcast_convert_type(
            s_lo + ((s_hi - s_lo) >> 1), jnp.float32)
        denom = jnp.maximum((c_lo - c_hi).astype(jnp.float32), 1.0)
        v_int = v_hi + (v_lo - v_hi) * (
            (jnp.float32(K) - c_hi.astype(jnp.float32)) / denom)
        use_int = ((it % 2) == 0) & (v_int > v_lo) & (v_int < v_hi)
        cand = jnp.where(done, thr, jnp.where(use_int, v_int, v_bis))

        cnt = _count_ge(pre, cand)

        hit = jnp.logical_not(done) & (cnt == K)
        thr = jnp.where(hit, cand, thr)
        done2 = done | hit
        golo = jnp.logical_not(done2) & (cnt > K)
        gohi = jnp.logical_not(done2) & (cnt < K)
        v_lo = jnp.where(golo, cand, v_lo)
        c_lo = jnp.where(golo, cnt, c_lo)
        v_hi = jnp.where(gohi, cand, v_hi)
        c_hi = jnp.where(gohi, cnt, c_hi)
        s_lo = jax.lax.bitcast_convert_type(v_lo, jnp.int32)
        s_hi = jax.lax.bitcast_convert_type(v_hi, jnp.int32)
        ex = jnp.logical_not(done2) & ((s_hi - s_lo) <= 1)
        thr = jnp.where(ex, v_lo, thr)
        done2 = done2 | ex
        return (it + 1, v_lo, v_hi, c_lo, c_hi, thr,
                done2.astype(jnp.int32))

    state = (it0, v_lo, v_hi, c_lo, c_hi, thr, done)
    state = jax.lax.while_loop(cond, body, state)
    thr = state[5]

    mask = (pre >= thr) & (pre > 0.0)
    z_ref[...] = jnp.where(mask, pre, 0.0)


def _decode_kernel(z_ref, x_ref, Wd_ref, bd_ref, xhat_ref, loss_ref):
    t = pl.program_id(0)
    rb = pl.program_id(1)
    xh = jax.lax.dot_general(
        z_ref[...], Wd_ref[0],
        (((1,), (1,)), ((), ())),
        preferred_element_type=jnp.float32,
    ) + bd_ref[0]
    xhat_ref[...] = xh
    err = xh - x_ref[...]

    @pl.when((t == 0) & (rb == 0))
    def _():
        loss_ref[...] = jnp.zeros((1, 1), jnp.float32)

    loss_ref[...] += jnp.sum(err * err).reshape(1, 1)


def kernel(x, W_enc, b_enc, W_dec, b_dec):
    B = x.shape[0]
    nb = B // RB
    grid = (T, nb)

    x2 = x.reshape(B, T * D_IN)
    be = b_enc.reshape(T, 1, D_SAE)
    bd = b_dec.reshape(T, 1, D_IN)

    z2 = pl.pallas_call(
        _encode_kernel,
        grid=grid,
        in_specs=[
            pl.BlockSpec((RB, D_IN), lambda t, rb: (rb, t)),
            pl.BlockSpec((1, D_IN, D_SAE), lambda t, rb: (t, 0, 0)),
            pl.BlockSpec((1, 1, D_SAE), lambda t, rb: (t, 0, 0)),
            pl.BlockSpec((1, 1, D_IN), lambda t, rb: (t, 0, 0)),
        ],
        out_specs=pl.BlockSpec((RB, D_SAE), lambda t, rb: (rb, t)),
        out_shape=jax.ShapeDtypeStruct((B, T * D_SAE), jnp.float32),
    )(x2, W_enc, be, bd)

    xhat2, loss_sum = pl.pallas_call(
        _decode_kernel,
        grid=grid,
        in_specs=[
            pl.BlockSpec((RB, D_SAE), lambda t, rb: (rb, t)),
            pl.BlockSpec((RB, D_IN), lambda t, rb: (rb, t)),
            pl.BlockSpec((1, D_IN, D_SAE), lambda t, rb: (t, 0, 0)),
            pl.BlockSpec((1, 1, D_IN), lambda t, rb: (t, 0, 0)),
        ],
        out_specs=[
            pl.BlockSpec((RB, D_IN), lambda t, rb: (rb, t)),
            pl.BlockSpec((1, 1), lambda t, rb: (0, 0)),
        ],
        out_shape=[
            jax.ShapeDtypeStruct((B, T * D_IN), jnp.float32),
            jax.ShapeDtypeStruct((1, 1), jnp.float32),
        ],
    )(z2, x2, W_dec, bd)

    loss = loss_sum[0, 0] / jnp.float32(B * T * D_IN)
    return (loss, xhat2.reshape(B, T, D_IN), z2.reshape(B, T, D_SAE))


# speedup vs baseline: 1.3800x; 1.3800x over previous
import jax
import jax.numpy as jnp
import numpy as np
from jax.experimental import pallas as pl
from jax.experimental.pallas import tpu as pltpu

D_IN = 1024
D_SAE = 4096
T = 8
K = 32
RB_E = 128
RB = 256

_SIGN = int(np.int32(np.uint32(0x80000000)))


def _encode_kernel(x_ref, We_ref, be_ref, bd_ref, z_ref):
    t = pl.program_id(0)
    x = x_ref[:, pl.ds(t, 1), :][:, 0, :]
    xc = x - bd_ref[0]
    pre = jax.lax.dot_general(
        xc, We_ref[0],
        (((1,), (0,)), ((), ())),
        preferred_element_type=jnp.float32,
    ) + be_ref[0]

    u = jax.lax.bitcast_convert_type(pre, jnp.int32)
    s = u ^ ((u >> 31) & jnp.int32(0x7FFFFFFF))

    prefix = jnp.zeros((x_ref.shape[0], 1), jnp.int32)
    for i in range(31, -1, -1):
        bit = int(np.int32(np.uint32(1 << i)))
        cand = prefix | bit
        cnt = jnp.sum((s >= (cand ^ _SIGN)).astype(jnp.int32), axis=1,
                      keepdims=True)
        prefix = jnp.where(cnt >= K, cand, prefix)
    tau = prefix ^ _SIGN

    mask = (s >= tau) & (pre > 0.0)
    z_ref[...] = jnp.where(mask, pre, 0.0)


def _decode_kernel(z_ref, x_ref, Wd_ref, bd_ref, xhat_ref, loss_ref):
    t = pl.program_id(0)
    rb = pl.program_id(1)
    xh = jax.lax.dot_general(
        z_ref[...], Wd_ref[0],
        (((1,), (1,)), ((), ())),
        preferred_element_type=jnp.float32,
    ) + bd_ref[0]
    xhat_ref[...] = xh
    err = xh - x_ref[:, pl.ds(t, 1), :][:, 0, :]

    @pl.when((t == 0) & (rb == 0))
    def _():
        loss_ref[...] = jnp.zeros((1, 1), jnp.float32)

    loss_ref[...] += jnp.sum(err * err).reshape(1, 1)


def kernel(x, W_enc, b_enc, W_dec, b_dec):
    B = x.shape[0]
    grid = (T, B // RB)

    be = b_enc.reshape(T, 1, D_SAE)
    bd = b_dec.reshape(T, 1, D_IN)

    z2 = pl.pallas_call(
        _encode_kernel,
        grid=(T, B // RB_E),
        in_specs=[
            pl.BlockSpec((RB_E, T, D_IN), lambda t, rb: (rb, 0, 0)),
            pl.BlockSpec((1, D_IN, D_SAE), lambda t, rb: (t, 0, 0)),
            pl.BlockSpec((1, 1, D_SAE), lambda t, rb: (t, 0, 0)),
            pl.BlockSpec((1, 1, D_IN), lambda t, rb: (t, 0, 0)),
        ],
        out_specs=pl.BlockSpec((RB_E, D_SAE), lambda t, rb: (rb, t)),
        out_shape=jax.ShapeDtypeStruct((B, T * D_SAE), jnp.float32),
        compiler_params=pltpu.CompilerParams(
            vmem_limit_bytes=62 * 1024 * 1024,
        ),
    )(x, W_enc, be, bd)

    xhat2, loss_sum = pl.pallas_call(
        _decode_kernel,
        grid=grid,
        in_specs=[
            pl.BlockSpec((RB, D_SAE), lambda t, rb: (rb, t)),
            pl.BlockSpec((RB, T, D_IN), lambda t, rb: (rb, 0, 0)),
            pl.BlockSpec((1, D_IN, D_SAE), lambda t, rb: (t, 0, 0)),
            pl.BlockSpec((1, 1, D_IN), lambda t, rb: (t, 0, 0)),
        ],
        out_specs=[
            pl.BlockSpec((RB, D_IN), lambda t, rb: (rb, t)),
            pl.BlockSpec((1, 1), lambda t, rb: (0, 0)),
        ],
        out_shape=[
            jax.ShapeDtypeStruct((B, T * D_IN), jnp.float32),
            jax.ShapeDtypeStruct((1, 1), jnp.float32),
        ],
        compiler_params=pltpu.CompilerParams(
            vmem_limit_bytes=62 * 1024 * 1024,
        ),
    )(z2, x, W_dec, bd)

    loss = loss_sum[0, 0] / jnp.float32(B * T * D_IN)
    return (loss, xhat2.reshape(B, T, D_IN), z2.reshape(B, T, D_SAE))


# speedup vs baseline: 1.4397x; 1.0433x over previous
import jax
import jax.numpy as jnp
import numpy as np
from jax.experimental import pallas as pl
from jax.experimental.pallas import tpu as pltpu

D_IN = 1024
D_SAE = 4096
T = 8
K = 32
RB = 256

_SIGN = int(np.int32(np.uint32(0x80000000)))


def _encode_kernel(x_ref, We_ref, be_ref, bd_ref, z_ref):
    xc = x_ref[...] - bd_ref[0]
    pre = jax.lax.dot_general(
        xc, We_ref[0],
        (((1,), (0,)), ((), ())),
        preferred_element_type=jnp.float32,
    ) + be_ref[0]

    u = jax.lax.bitcast_convert_type(pre, jnp.int32)
    s = u ^ ((u >> 31) & jnp.int32(0x7FFFFFFF))

    prefix = jnp.zeros((x_ref.shape[0], 1), jnp.int32)
    for i in range(31, -1, -1):
        bit = int(np.int32(np.uint32(1 << i)))
        cand = prefix | bit
        cnt = jnp.sum(s >= (cand ^ _SIGN), axis=1,
                      keepdims=True, dtype=jnp.int32)
        prefix = jnp.where(cnt >= K, cand, prefix)
    tau = prefix ^ _SIGN

    mask = (s >= tau) & (pre > 0.0)
    z_ref[...] = jnp.where(mask, pre, 0.0)


def _decode_kernel(z_ref, x_ref, Wd_ref, bd_ref, xhat_ref, loss_ref):
    t = pl.program_id(0)
    rb = pl.program_id(1)
    xh = jax.lax.dot_general(
        z_ref[...], Wd_ref[0],
        (((1,), (1,)), ((), ())),
        preferred_element_type=jnp.float32,
    ) + bd_ref[0]
    xhat_ref[...] = xh
    err = xh - x_ref[...]

    @pl.when((t == 0) & (rb == 0))
    def _():
        loss_ref[...] = jnp.zeros((1, 1), jnp.float32)

    loss_ref[...] += jnp.sum(err * err).reshape(1, 1)


def kernel(x, W_enc, b_enc, W_dec, b_dec):
    B = x.shape[0]
    nb = B // RB
    grid = (T, nb)

    x2 = x.reshape(B, T * D_IN)
    be = b_enc.reshape(T, 1, D_SAE)
    bd = b_dec.reshape(T, 1, D_IN)

    z2 = pl.pallas_call(
        _encode_kernel,
        grid=grid,
        in_specs=[
            pl.BlockSpec((RB, D_IN), lambda t, rb: (rb, t)),
            pl.BlockSpec((1, D_IN, D_SAE), lambda t, rb: (t, 0, 0)),
            pl.BlockSpec((1, 1, D_SAE), lambda t, rb: (t, 0, 0)),
            pl.BlockSpec((1, 1, D_IN), lambda t, rb: (t, 0, 0)),
        ],
        out_specs=pl.BlockSpec((RB, D_SAE), lambda t, rb: (rb, t)),
        out_shape=jax.ShapeDtypeStruct((B, T * D_SAE), jnp.float32),
        compiler_params=pltpu.CompilerParams(
            vmem_limit_bytes=62 * 1024 * 1024,
        ),
    )(x2, W_enc, be, bd)

    xhat2, loss_sum = pl.pallas_call(
        _decode_kernel,
        grid=grid,
        in_specs=[
            pl.BlockSpec((RB, D_SAE), lambda t, rb: (rb, t)),
            pl.BlockSpec((RB, D_IN), lambda t, rb: (rb, t)),
            pl.BlockSpec((1, D_IN, D_SAE), lambda t, rb: (t, 0, 0)),
            pl.BlockSpec((1, 1, D_IN), lambda t, rb: (t, 0, 0)),
        ],
        out_specs=[
            pl.BlockSpec((RB, D_IN), lambda t, rb: (rb, t)),
            pl.BlockSpec((1, 1), lambda t, rb: (0, 0)),
        ],
        out_shape=[
            jax.ShapeDtypeStruct((B, T * D_IN), jnp.float32),
            jax.ShapeDtypeStruct((1, 1), jnp.float32),
        ],
        compiler_params=pltpu.CompilerParams(
            vmem_limit_bytes=62 * 1024 * 1024,
        ),
    )(z2, x2, W_dec, bd)

    loss = loss_sum[0, 0] / jnp.float32(B * T * D_IN)
    return (loss, xhat2.reshape(B, T, D_IN), z2.reshape(B, T, D_SAE))
